# baseline (speedup 1.0000x reference)
import jax
import jax.numpy as jnp
from jax import lax
from jax.experimental import pallas as pl
from jax.experimental.pallas import tpu as pltpu

HIDDEN = 4096
NUM_EXPERTS = 64
NUM_TOKENS = 32768

BT = 512
NBUF = 4

_DN = (((1,), (1,)), ((), ()))


def _body(x_hbm, w_ref, b_ref, o_hbm):
    def inner(x_blk, o_blk):
        o_blk[...] = (
            lax.dot_general(x_blk[...], w_ref[...], _DN,
                            preferred_element_type=jnp.float32)
            + b_ref[...]
        )

    pltpu.emit_pipeline(
        inner,
        grid=(NUM_TOKENS // BT,),
        in_specs=[
            pl.BlockSpec((BT, HIDDEN), lambda i: (i, 0),
                         pipeline_mode=pl.Buffered(buffer_count=NBUF)),
        ],
        out_specs=[
            pl.BlockSpec((BT, NUM_EXPERTS), lambda i: (i, 0)),
        ],
    )(x_hbm, o_hbm)


def kernel(x, W, b):
    b2 = b.reshape(1, NUM_EXPERTS)
    return pl.pallas_call(
        _body,
        in_specs=[
            pl.BlockSpec(memory_space=pltpu.HBM),
            pl.BlockSpec((NUM_EXPERTS, HIDDEN), lambda: (0, 0)),
            pl.BlockSpec((1, NUM_EXPERTS), lambda: (0, 0)),
        ],
        out_specs=pl.BlockSpec(memory_space=pltpu.HBM),
        out_shape=jax.ShapeDtypeStruct((NUM_TOKENS, NUM_EXPERTS), jnp.float32),
    )(x, W, b2)


# speedup vs baseline: 1.0469x; 1.0469x over previous
import jax
import jax.numpy as jnp
from jax import lax
from jax.experimental import pallas as pl
from jax.experimental.pallas import tpu as pltpu

HIDDEN = 4096
NUM_EXPERTS = 64
NUM_TOKENS = 32768

BT = 512

_DN = (((1,), (1,)), ((), ()))


def _router_block(x_ref, w_ref, b_ref, o_ref):
    xb = x_ref[...].astype(jnp.bfloat16)
    wb = w_ref[...].astype(jnp.bfloat16)
    o_ref[...] = (
        lax.dot_general(xb, wb, _DN, preferred_element_type=jnp.float32)
        + b_ref[...]
    )


def kernel(x, W, b):
    b2 = b.reshape(1, NUM_EXPERTS)
    grid = (NUM_TOKENS // BT,)
    return pl.pallas_call(
        _router_block,
        grid=grid,
        in_specs=[
            pl.BlockSpec((BT, HIDDEN), lambda i: (i, 0)),
            pl.BlockSpec((NUM_EXPERTS, HIDDEN), lambda i: (0, 0)),
            pl.BlockSpec((1, NUM_EXPERTS), lambda i: (0, 0)),
        ],
        out_specs=pl.BlockSpec((BT, NUM_EXPERTS), lambda i: (i, 0)),
        out_shape=jax.ShapeDtypeStruct((NUM_TOKENS, NUM_EXPERTS), jnp.float32),
        compiler_params=pltpu.CompilerParams(
            dimension_semantics=("parallel",),
        ),
    )(x, W, b2)
